# baseline (speedup 1.0000x reference)
import functools

import jax
import jax.numpy as jnp
from jax import lax
from jax.experimental import pallas as pl
from jax.experimental.pallas import tpu as pltpu
from jax.experimental.pallas import tpu_sc as plsc

N = 10000
E = 320000
D = 128
NC = 2
NS = 16
NW = NC * NS
CH = 80
EPW = E // NW
RPW = EPW // CH
RPW2 = E // NS // CH
ZCH = 200
NIO = 10

f32 = jnp.float32
i32 = jnp.int32


def _sc_mesh():
    return plsc.VectorSubcoreMesh(
        core_axis_name="c", subcore_axis_name="s", num_cores=NC, num_subcores=NS
    )


def _deg_partials(dst3, ew3):
    @functools.partial(
        pl.kernel,
        out_type=jax.ShapeDtypeStruct((NC * N,), f32),
        mesh=_sc_mesh(),
        compiler_params=pltpu.CompilerParams(use_tc_tiling_on_sc=False),
        scratch_types=[
            pltpu.VMEM((RPW, CH), i32),
            pltpu.VMEM((RPW, CH), f32),
            pltpu.VMEM((1024,), f32),
            pltpu.VMEM_SHARED((N,), f32),
        ],
    )
    def k(dst_hbm, ew_hbm, deg_hbm, idx_v, ew_v, zbuf, deg_sh):
        c = lax.axis_index("c")
        s = lax.axis_index("s")
        wid = c * NS + s
        pltpu.sync_copy(dst_hbm.at[wid], idx_v)
        pltpu.sync_copy(ew_hbm.at[wid], ew_v)
        z = jnp.zeros((16,), f32)
        for t in range(64):
            zbuf[pl.ds(t * 16, 16)] = z
        @pl.when(s < NIO)
        def _():
            pltpu.sync_copy(zbuf.at[pl.ds(0, 1000)], deg_sh.at[pl.ds(s * 1000, 1000)])

        plsc.subcore_barrier()

        @pl.loop(0, RPW)
        def _(i):
            pltpu.sync_copy(ew_v.at[i], deg_sh.at[idx_v.at[i]], add=True)

        plsc.subcore_barrier()

        @pl.when(s < NIO)
        def _():
            pltpu.sync_copy(deg_sh.at[pl.ds(s * 1000, 1000)], zbuf.at[pl.ds(0, 1000)])
            pltpu.sync_copy(
                zbuf.at[pl.ds(0, 1000)],
                deg_hbm.at[pl.ds(c * N + s * 1000, 1000)],
            )

    return k(dst3, ew3)


D2 = D // 2


def _msg_partials(src16, dst16, ew16, h2cat):
    @functools.partial(
        pl.kernel,
        out_type=jax.ShapeDtypeStruct((NC, N, D2), f32),
        mesh=_sc_mesh(),
        compiler_params=pltpu.CompilerParams(
            use_tc_tiling_on_sc=False, needs_layout_passes=False
        ),
        scratch_types=[
            pltpu.VMEM((RPW2, CH), i32),
            pltpu.VMEM((RPW2, CH), i32),
            pltpu.VMEM((RPW2, CH), f32),
            pltpu.VMEM((CH, D2), jnp.bfloat16),
            pltpu.VMEM((CH, D2), jnp.bfloat16),
            pltpu.VMEM((CH, D2), f32),
            pltpu.VMEM_SHARED((N, D2), f32),
            pltpu.SemaphoreType.DMA,
            pltpu.SemaphoreType.DMA,
        ],
    )
    def k(src_hbm, dst_hbm, ew_hbm, h2_hbm, out_hbm, src_v, dst_v, ew_v,
          rows0, rows1, fbuf, out_sh, sem0, sem1):
        c = lax.axis_index("c")
        s = lax.axis_index("s")
        pltpu.sync_copy(src_hbm.at[s], src_v)
        pltpu.sync_copy(dst_hbm.at[s], dst_v)
        pltpu.sync_copy(ew_hbm.at[s], ew_v)

        off = jnp.full((16,), c * N, i32)

        @pl.loop(0, RPW2)
        def _(r):
            for cc in range(CH // 16):
                src_v[r, pl.ds(cc * 16, 16)] = src_v[r, pl.ds(cc * 16, 16)] + off

        z = jnp.zeros((16,), f32)

        @pl.loop(0, CH)
        def _(r):
            for cc in range(D2 // 16):
                fbuf[r, pl.ds(cc * 16, 16)] = z

        base = s * 1000

        @pl.when(s < NIO)
        def _():
            for j in range(12):
                pltpu.sync_copy(fbuf, out_sh.at[pl.ds(base + j * CH, CH)])
            pltpu.sync_copy(fbuf.at[pl.ds(0, 40)], out_sh.at[pl.ds(base + 960, 40)])

        plsc.subcore_barrier()

        def scale(buf, i):
            for g in range(CH // 16):
                wv = ew_v[i, pl.ds(g * 16, 16)]
                for j2 in range(16):
                    w = wv[j2]
                    r = g * 16 + j2
                    for cc in range(2):
                        v = buf[r, pl.ds(cc * 32, 32)]
                        a, bq = plsc.unpack(v, format=plsc.PackFormat.INTERLEAVED)
                        fbuf[r, pl.ds(cc * 16, 16)] = a * w
                        fbuf[r, pl.ds(32 + cc * 16, 16)] = bq * w

        bufs = (rows0, rows1)
        sems = (sem0, sem1)
        for b in range(2):
            pltpu.async_copy(h2_hbm.at[src_v.at[b]], bufs[b], sems[b])

        @pl.loop(0, RPW2 - 2, step=2)
        def _(i):
            for b in range(2):
                ch = i + b
                pltpu.make_async_copy(h2_hbm.at[src_v.at[ch]], bufs[b], sems[b]).wait()
                scale(bufs[b], ch)
                pltpu.async_copy(h2_hbm.at[src_v.at[ch + 2]], bufs[b], sems[b])
                pltpu.sync_copy(fbuf, out_sh.at[dst_v.at[ch]], add=True)

        for t in range(2):
            ch_t = RPW2 - 2 + t
            pltpu.make_async_copy(h2_hbm.at[src_v.at[ch_t]], bufs[t], sems[t]).wait()
            scale(bufs[t], ch_t)
            pltpu.sync_copy(fbuf, out_sh.at[dst_v.at[ch_t]], add=True)

        plsc.subcore_barrier()

        @pl.when(s < NIO)
        def _():
            for j in range(12):
                pltpu.sync_copy(out_sh.at[pl.ds(base + j * CH, CH)], fbuf)
                pltpu.sync_copy(fbuf, out_hbm.at[c, pl.ds(base + j * CH, CH)])
            pltpu.sync_copy(out_sh.at[pl.ds(base + 960, 40)], fbuf.at[pl.ds(0, 40)])
            pltpu.sync_copy(fbuf.at[pl.ds(0, 40)], out_hbm.at[c, pl.ds(base + 960, 40)])

    return k(src16, dst16, ew16, h2cat)


_BLK = 1000


def _linear_norm(x, W, dega2, degb2):
    def body(x_ref, w_ref, da_ref, db_ref, h2cat_ref, dis_ref):
        dis = lax.rsqrt(1.0 + da_ref[...] + db_ref[...])
        h = jnp.dot(x_ref[...], w_ref[...], preferred_element_type=f32)
        h2 = h * dis

        def ileave(v):
            return jnp.stack([v[:, : D2 // 2], v[:, D2 // 2 :]], axis=-1).reshape(
                v.shape[0], D2
            )

        h2cat_ref[0] = ileave(h2[:, :D2]).astype(jnp.bfloat16)
        h2cat_ref[1] = ileave(h2[:, D2:]).astype(jnp.bfloat16)
        dis_ref[...] = dis

    return pl.pallas_call(
        body,
        grid=(N // _BLK,),
        in_specs=[
            pl.BlockSpec((_BLK, D), lambda i: (i, 0)),
            pl.BlockSpec((D, D), lambda i: (0, 0)),
            pl.BlockSpec((_BLK, 1), lambda i: (i, 0)),
            pl.BlockSpec((_BLK, 1), lambda i: (i, 0)),
        ],
        out_specs=[
            pl.BlockSpec((2, _BLK, D2), lambda i: (0, i, 0)),
            pl.BlockSpec((_BLK, 1), lambda i: (i, 0)),
        ],
        out_shape=[
            jax.ShapeDtypeStruct((2, N, D2), jnp.bfloat16),
            jax.ShapeDtypeStruct((N, 1), f32),
        ],
    )(x, W, dega2, degb2)


def _combine(p, h2cat, dis2, b2):
    def body(pa_ref, pb_ref, h2a_ref, h2b_ref, dis_ref, b_ref, o_ref):
        dis = dis_ref[...]
        b_blk = b_ref[...]

        def deileave(v):
            w = v.reshape(v.shape[0], D2 // 2, 2)
            return jnp.concatenate([w[:, :, 0], w[:, :, 1]], axis=1)

        h2a = deileave(h2a_ref[0].astype(f32))
        h2b = deileave(h2b_ref[0].astype(f32))
        acca = pa_ref[0] + h2a
        accb = pb_ref[0] + h2b
        o_ref[:, :D2] = jnp.maximum(dis * acca + b_blk[:, :D2], 0.0)
        o_ref[:, D2:] = jnp.maximum(dis * accb + b_blk[:, D2:], 0.0)

    return pl.pallas_call(
        body,
        grid=(N // _BLK,),
        in_specs=[
            pl.BlockSpec((1, _BLK, D2), lambda i: (0, i, 0)),
            pl.BlockSpec((1, _BLK, D2), lambda i: (1, i, 0)),
            pl.BlockSpec((1, _BLK, D2), lambda i: (0, i, 0)),
            pl.BlockSpec((1, _BLK, D2), lambda i: (1, i, 0)),
            pl.BlockSpec((_BLK, 1), lambda i: (i, 0)),
            pl.BlockSpec((1, D), lambda i: (0, 0)),
        ],
        out_specs=pl.BlockSpec((_BLK, D), lambda i: (i, 0)),
        out_shape=jax.ShapeDtypeStruct((N, D), f32),
    )(p, p, h2cat, h2cat, dis2, b2)


def kernel(x, edge_index, edge_weight, W, b):
    src_e = edge_index[0].astype(i32)
    dst_e = edge_index[1].astype(i32)
    ew_e = edge_weight.astype(f32)

    degs = _deg_partials(
        dst_e.reshape(NW, RPW, CH), ew_e.reshape(NW, RPW, CH)
    )
    h2cat, dis2 = _linear_norm(
        x, W, degs[:N].reshape(N, 1), degs[N:].reshape(N, 1)
    )
    p = _msg_partials(
        src_e.reshape(NS, RPW2, CH),
        dst_e.reshape(NS, RPW2, CH),
        ew_e.reshape(NS, RPW2, CH),
        h2cat.reshape(2 * N, D2),
    )
    return _combine(p, h2cat, dis2, b.reshape(1, D))


# speedup vs baseline: 2.9523x; 2.9523x over previous
import functools

import jax
import jax.numpy as jnp
from jax import lax
from jax.experimental import pallas as pl
from jax.experimental.pallas import tpu as pltpu
from jax.experimental.pallas import tpu_sc as plsc

N = 10000
E = 320000
D = 128
NC = 2
NS = 16
NW = NC * NS
CH = 80
EPW = E // NW
RPW = EPW // CH
RPW2 = E // NS // CH
ZCH = 200
NIO = 10

f32 = jnp.float32
i32 = jnp.int32


def _sc_mesh():
    return plsc.VectorSubcoreMesh(
        core_axis_name="c", subcore_axis_name="s", num_cores=NC, num_subcores=NS
    )


def _deg_partials(dst3, ew3):
    @functools.partial(
        pl.kernel,
        out_type=jax.ShapeDtypeStruct((NC * N,), f32),
        mesh=_sc_mesh(),
        compiler_params=pltpu.CompilerParams(use_tc_tiling_on_sc=False),
        scratch_types=[
            pltpu.VMEM((RPW, CH), i32),
            pltpu.VMEM((RPW, CH), f32),
            pltpu.VMEM((1024,), f32),
            pltpu.VMEM_SHARED((N,), f32),
        ],
    )
    def k(dst_hbm, ew_hbm, deg_hbm, idx_v, ew_v, zbuf, deg_sh):
        c = lax.axis_index("c")
        s = lax.axis_index("s")
        wid = c * NS + s
        pltpu.sync_copy(dst_hbm.at[wid], idx_v)
        pltpu.sync_copy(ew_hbm.at[wid], ew_v)
        z = jnp.zeros((16,), f32)
        for t in range(64):
            zbuf[pl.ds(t * 16, 16)] = z
        @pl.when(s < NIO)
        def _():
            pltpu.sync_copy(zbuf.at[pl.ds(0, 1000)], deg_sh.at[pl.ds(s * 1000, 1000)])

        plsc.subcore_barrier()

        @pl.loop(0, RPW)
        def _(i):
            pltpu.sync_copy(ew_v.at[i], deg_sh.at[idx_v.at[i]], add=True)

        plsc.subcore_barrier()

        @pl.when(s < NIO)
        def _():
            pltpu.sync_copy(deg_sh.at[pl.ds(s * 1000, 1000)], zbuf.at[pl.ds(0, 1000)])
            pltpu.sync_copy(
                zbuf.at[pl.ds(0, 1000)],
                deg_hbm.at[pl.ds(c * N + s * 1000, 1000)],
            )

    return k(dst3, ew3)


D2 = D // 2


def _msg_partials(src16, dst16, ew16, h2cat):
    @functools.partial(
        pl.kernel,
        out_type=jax.ShapeDtypeStruct((NC, N, D2), f32),
        mesh=_sc_mesh(),
        compiler_params=pltpu.CompilerParams(use_tc_tiling_on_sc=False),
        scratch_types=[
            pltpu.VMEM((RPW2, CH), i32),
            pltpu.VMEM((RPW2, CH), i32),
            pltpu.VMEM((RPW2, CH), f32),
            pltpu.VMEM((CH, D2), f32),
            pltpu.VMEM((CH, D2), f32),
            pltpu.VMEM((CH, D2), f32),
            pltpu.VMEM_SHARED((N, D2), f32),
            pltpu.SemaphoreType.DMA,
            pltpu.SemaphoreType.DMA,
            pltpu.SemaphoreType.DMA,
        ],
    )
    def k(src_hbm, dst_hbm, ew_hbm, h2_hbm, out_hbm, src_v, dst_v, ew_v,
          rows0, rows1, rows2, out_sh, sem0, sem1, sem2):
        c = lax.axis_index("c")
        s = lax.axis_index("s")
        pltpu.sync_copy(src_hbm.at[s], src_v)
        pltpu.sync_copy(dst_hbm.at[s], dst_v)
        pltpu.sync_copy(ew_hbm.at[s], ew_v)

        off = jnp.full((16,), c * N, i32)

        @pl.loop(0, RPW2)
        def _(r):
            for cc in range(CH // 16):
                src_v[r, pl.ds(cc * 16, 16)] = src_v[r, pl.ds(cc * 16, 16)] + off

        z = jnp.zeros((16,), f32)

        @pl.loop(0, CH)
        def _(r):
            for cc in range(D2 // 16):
                rows0[r, pl.ds(cc * 16, 16)] = z

        base = s * 1000

        @pl.when(s < NIO)
        def _():
            for j in range(12):
                pltpu.sync_copy(rows0, out_sh.at[pl.ds(base + j * CH, CH)])
            pltpu.sync_copy(rows0.at[pl.ds(0, 40)], out_sh.at[pl.ds(base + 960, 40)])

        plsc.subcore_barrier()

        def scale(buf, i):
            for g in range(CH // 16):
                wv = ew_v[i, pl.ds(g * 16, 16)]
                for j2 in range(16):
                    w = wv[j2]
                    r = g * 16 + j2
                    for cc in range(D2 // 16):
                        buf[r, pl.ds(cc * 16, 16)] = buf[r, pl.ds(cc * 16, 16)] * w

        bufs = (rows0, rows1, rows2)
        sems = (sem0, sem1, sem2)
        for b in range(3):
            pltpu.async_copy(h2_hbm.at[src_v.at[b]], bufs[b], sems[b])

        @pl.loop(0, RPW2 - 4, step=3)
        def _(i):
            for b in range(3):
                ch = i + b
                pltpu.make_async_copy(h2_hbm.at[src_v.at[ch]], bufs[b], sems[b]).wait()
                scale(bufs[b], ch)
                pltpu.sync_copy(bufs[b], out_sh.at[dst_v.at[ch]], add=True)
                pltpu.async_copy(h2_hbm.at[src_v.at[ch + 3]], bufs[b], sems[b])

        for t in range(4):
            ch_t = RPW2 - 4 + t
            b_t = ch_t % 3
            pltpu.make_async_copy(h2_hbm.at[src_v.at[ch_t]], bufs[b_t], sems[b_t]).wait()
            scale(bufs[b_t], ch_t)
            pltpu.sync_copy(bufs[b_t], out_sh.at[dst_v.at[ch_t]], add=True)

            @pl.when(ch_t + 3 < RPW2)
            def _():
                pltpu.async_copy(h2_hbm.at[src_v.at[ch_t + 3]], bufs[b_t], sems[b_t])

        plsc.subcore_barrier()

        @pl.when(s < NIO)
        def _():
            for j in range(12):
                pltpu.sync_copy(out_sh.at[pl.ds(base + j * CH, CH)], rows0)
                pltpu.sync_copy(rows0, out_hbm.at[c, pl.ds(base + j * CH, CH)])
            pltpu.sync_copy(out_sh.at[pl.ds(base + 960, 40)], rows0.at[pl.ds(0, 40)])
            pltpu.sync_copy(rows0.at[pl.ds(0, 40)], out_hbm.at[c, pl.ds(base + 960, 40)])

    return k(src16, dst16, ew16, h2cat)


_BLK = 1000


def _linear_norm(x, W, dega2, degb2):
    def body(x_ref, w_ref, da_ref, db_ref, h2cat_ref, dis_ref):
        dis = lax.rsqrt(1.0 + da_ref[...] + db_ref[...])
        h = jnp.dot(x_ref[...], w_ref[...], preferred_element_type=f32)
        h2 = h * dis
        h2cat_ref[0] = h2[:, :D2]
        h2cat_ref[1] = h2[:, D2:]
        dis_ref[...] = dis

    return pl.pallas_call(
        body,
        grid=(N // _BLK,),
        in_specs=[
            pl.BlockSpec((_BLK, D), lambda i: (i, 0)),
            pl.BlockSpec((D, D), lambda i: (0, 0)),
            pl.BlockSpec((_BLK, 1), lambda i: (i, 0)),
            pl.BlockSpec((_BLK, 1), lambda i: (i, 0)),
        ],
        out_specs=[
            pl.BlockSpec((2, _BLK, D2), lambda i: (0, i, 0)),
            pl.BlockSpec((_BLK, 1), lambda i: (i, 0)),
        ],
        out_shape=[
            jax.ShapeDtypeStruct((2, N, D2), f32),
            jax.ShapeDtypeStruct((N, 1), f32),
        ],
    )(x, W, dega2, degb2)


def _combine(p, h2cat, dis2, b2):
    def body(pa_ref, pb_ref, h2a_ref, h2b_ref, dis_ref, b_ref, o_ref):
        dis = dis_ref[...]
        b_blk = b_ref[...]
        acca = pa_ref[0] + h2a_ref[0]
        accb = pb_ref[0] + h2b_ref[0]
        o_ref[:, :D2] = jnp.maximum(dis * acca + b_blk[:, :D2], 0.0)
        o_ref[:, D2:] = jnp.maximum(dis * accb + b_blk[:, D2:], 0.0)

    return pl.pallas_call(
        body,
        grid=(N // _BLK,),
        in_specs=[
            pl.BlockSpec((1, _BLK, D2), lambda i: (0, i, 0)),
            pl.BlockSpec((1, _BLK, D2), lambda i: (1, i, 0)),
            pl.BlockSpec((1, _BLK, D2), lambda i: (0, i, 0)),
            pl.BlockSpec((1, _BLK, D2), lambda i: (1, i, 0)),
            pl.BlockSpec((_BLK, 1), lambda i: (i, 0)),
            pl.BlockSpec((1, D), lambda i: (0, 0)),
        ],
        out_specs=pl.BlockSpec((_BLK, D), lambda i: (i, 0)),
        out_shape=jax.ShapeDtypeStruct((N, D), f32),
    )(p, p, h2cat, h2cat, dis2, b2)


def kernel(x, edge_index, edge_weight, W, b):
    src_e = edge_index[0].astype(i32)
    dst_e = edge_index[1].astype(i32)
    ew_e = edge_weight.astype(f32)

    degs = _deg_partials(
        dst_e.reshape(NW, RPW, CH), ew_e.reshape(NW, RPW, CH)
    )
    h2cat, dis2 = _linear_norm(
        x, W, degs[:N].reshape(N, 1), degs[N:].reshape(N, 1)
    )
    p = _msg_partials(
        src_e.reshape(NS, RPW2, CH),
        dst_e.reshape(NS, RPW2, CH),
        ew_e.reshape(NS, RPW2, CH),
        h2cat.reshape(2 * N, D2),
    )
    return _combine(p, h2cat, dis2, b.reshape(1, D))
